# baseline (speedup 1.0000x reference)
import functools

import jax
import jax.numpy as jnp
import numpy as np
from jax import lax
from jax.experimental import pallas as pl
from jax.experimental.pallas import tpu as pltpu
from jax.experimental.pallas import tpu_sc as plsc

_N = 16384
_HID = 256
_OUT = 128
_NH = 16
_NB = 1024

_MIX = np.zeros((1, _OUT), dtype=np.int32)
_MIX[0, :_NH] = np.array(
    [73856093, 19349663, 83492791, 49979687, 67867967, 86028121,
     15485863, 32452843, 49979693, 67867979, 86028157, 15485867,
     2654435761 % (2**31 - 1), 40503, 2246822519 % (2**31 - 1),
     3266489917 % (2**31 - 1)], dtype=np.int32)


_BLK = 4096
_GRID = _N // _BLK


def _tc_body(x_ref, w1_ref, b1_ref, w2_ref, b2_ref, al_ref, bl_ref, pr_ref,
             out_ref, bkt_ref):
    x = x_ref[...]
    h = lax.dot_general(x, w1_ref[...], (((1,), (0,)), ((), ())),
                        preferred_element_type=jnp.float32)
    h = jnp.maximum(h + b1_ref[...], 0.0)
    out = lax.dot_general(h, w2_ref[...], (((1,), (0,)), ((), ())),
                          preferred_element_type=jnp.float32)
    out = out + b2_ref[...]
    out_ref[...] = out
    proj = lax.dot_general(out, al_ref[...], (((1,), (0,)), ((), ())),
                           preferred_element_type=jnp.float32)
    proj = proj + bl_ref[...]
    hcodes = jnp.floor(proj * 0.25).astype(jnp.int32)
    mixed = jnp.sum(hcodes * pr_ref[...], axis=1)
    bkt_ref[...] = lax.bitwise_and(mixed, 1023).reshape(_BLK // _C, _C)


_tc_call = pl.pallas_call(
    _tc_body,
    grid=(_GRID,),
    in_specs=[
        pl.BlockSpec((_BLK, _HID), lambda i: (i, 0)),
        pl.BlockSpec((_HID, _HID), lambda i: (0, 0)),
        pl.BlockSpec((1, _HID), lambda i: (0, 0)),
        pl.BlockSpec((_HID, _OUT), lambda i: (0, 0)),
        pl.BlockSpec((1, _OUT), lambda i: (0, 0)),
        pl.BlockSpec((_OUT, _OUT), lambda i: (0, 0)),
        pl.BlockSpec((1, _OUT), lambda i: (0, 0)),
        pl.BlockSpec((1, _OUT), lambda i: (0, 0)),
    ],
    out_specs=[
        pl.BlockSpec((_BLK, _OUT), lambda i: (i, 0)),
        pl.BlockSpec((_BLK // 128, 128), lambda i: (i, 0)),
    ],
    out_shape=[
        jax.ShapeDtypeStruct((_N, _OUT), jnp.float32),
        jax.ShapeDtypeStruct((_N // 128, 128), jnp.int32),
    ],
)


_NC = 2
_NS = 16
_PT = _N // (_NC * _NS)
_BT = _NB // _NS
_C = 128
_NCH = _PT // _C


@functools.cache
def _build_sc_sums():
  mesh = plsc.VectorSubcoreMesh(core_axis_name="c", subcore_axis_name="s",
                                num_cores=_NC, num_subcores=_NS)

  @functools.partial(
      pl.kernel,
      out_type=(jax.ShapeDtypeStruct((_NC, _NB, _OUT), jnp.float32),
                jax.ShapeDtypeStruct((_NC, _NB), jnp.float32)),
      mesh=mesh,
      compiler_params=pltpu.CompilerParams(needs_layout_passes=False),
      scratch_types=[
          pltpu.VMEM((_C, _OUT), jnp.float32),
          pltpu.VMEM((_C, _OUT), jnp.float32),
          pltpu.VMEM((_C, _OUT), jnp.float32),
          pltpu.VMEM((_C, _OUT), jnp.float32),
          pltpu.VMEM((2 * _NCH, _C), jnp.int32),
          pltpu.VMEM((_C,), jnp.float32),
          pltpu.VMEM((_BT, _OUT), jnp.float32),
          pltpu.VMEM((_BT,), jnp.float32),
          pltpu.VMEM_SHARED((_NB, _OUT), jnp.float32),
          pltpu.VMEM_SHARED((_NB,), jnp.float32),
          pltpu.SemaphoreType.DMA,
          pltpu.SemaphoreType.DMA,
          pltpu.SemaphoreType.DMA,
          pltpu.SemaphoreType.DMA,
          pltpu.SemaphoreType.DMA,
          pltpu.SemaphoreType.DMA,
          pltpu.SemaphoreType.DMA,
          pltpu.SemaphoreType.DMA,
          pltpu.SemaphoreType.DMA,
      ],
  )
  def _sums_call(out_hbm, bkt_hbm, sums2_hbm, cnt2_hbm,
                 buf0, buf1, buf2, buf3, idx_all, ones_v, work_v, cnt_v,
                 sums_sh, cnt_sh,
                 ld0, ld1, ld2, ld3, sc0, sc1, sc2, sc3, csem):
    c = lax.axis_index("c")
    t = lax.axis_index("s")
    base = c * (_N // _NC) + t * _PT
    brow = t * _BT
    bufs = (buf0, buf1, buf2, buf3)
    ldsems = (ld0, ld1, ld2, ld3)
    scsems = (sc0, sc1, sc2, sc3)

    zeros16 = jnp.zeros((16,), jnp.float32)
    for g in range(8):
      ones_v[pl.ds(g * 16, 16)] = jnp.ones((16,), jnp.float32)
    for g in range(_BT // 16):
      cnt_v[pl.ds(g * 16, 16)] = zeros16

    def _zrow(r, carry):
      for g in range(8):
        work_v[r, pl.ds(g * 16, 16)] = zeros16
      return carry
    lax.fori_loop(0, _BT, _zrow, 0)

    arow = pl.multiple_of(c * (_NB // _NS) + (t // 2) * (2 * _NCH), 8)
    half = (t % 2) * _NCH
    pltpu.sync_copy(bkt_hbm.at[pl.ds(arow, 2 * _NCH)], idx_all)

    pltpu.sync_copy(work_v, sums_sh.at[pl.ds(brow, _BT)])
    pltpu.sync_copy(cnt_v.at[pl.ds(0, _BT)], cnt_sh.at[pl.ds(brow, _BT)])
    plsc.subcore_barrier()

    ch = [pltpu.async_copy(ones_v, cnt_sh.at[idx_all.at[half + k]],
                           csem, add=True) for k in range(_NCH)]

    ldh = [pltpu.async_copy(out_hbm.at[pl.ds(base + k * _C, _C)],
                            bufs[k], ldsems[k]) for k in range(_NCH)]
    sch = [None] * _NCH
    for k in range(_NCH):
      ldh[k].wait()
      sch[k] = pltpu.async_copy(bufs[k], sums_sh.at[idx_all.at[half + k]],
                                scsems[k], add=True)
    for k in range(_NCH):
      sch[k].wait()
      ch[k].wait()
    plsc.subcore_barrier()

    pltpu.sync_copy(sums_sh.at[pl.ds(brow, _BT)], work_v)
    pltpu.sync_copy(cnt_sh.at[pl.ds(brow, _BT)], cnt_v)
    pltpu.sync_copy(work_v, sums2_hbm.at[c, pl.ds(brow, _BT)])
    pltpu.sync_copy(cnt_v, cnt2_hbm.at[c, pl.ds(brow, _BT)])

  return _sums_call


@functools.cache
def _build_sc_dists():
  mesh = plsc.VectorSubcoreMesh(core_axis_name="c", subcore_axis_name="s",
                                num_cores=_NC, num_subcores=_NS)

  @functools.partial(
      pl.kernel,
      out_type=(jax.ShapeDtypeStruct((_NC, _NB), jnp.float32),
                jax.ShapeDtypeStruct((_NC, _NB, _OUT), jnp.float32)),
      mesh=mesh,
      compiler_params=pltpu.CompilerParams(needs_layout_passes=False),
      scratch_types=[
          pltpu.VMEM((_C, _OUT), jnp.float32),
          pltpu.VMEM((_C, _OUT), jnp.float32),
          pltpu.VMEM((_C, _OUT), jnp.float32),
          pltpu.VMEM((_C, _OUT), jnp.float32),
          pltpu.VMEM((_C, _OUT), jnp.float32),
          pltpu.VMEM((_C, _OUT), jnp.float32),
          pltpu.VMEM((2 * _NCH, _C), jnp.int32),
          pltpu.VMEM((_C,), jnp.float32),
          pltpu.VMEM((_C,), jnp.float32),
          pltpu.VMEM((_BT, _OUT), jnp.float32),
          pltpu.VMEM((_BT,), jnp.float32),
          pltpu.VMEM((_BT,), jnp.float32),
          pltpu.VMEM((16, 17), jnp.float32),
          pltpu.VMEM_SHARED((_NB,), jnp.float32),
          pltpu.SemaphoreType.DMA,
          pltpu.SemaphoreType.DMA,
          pltpu.SemaphoreType.DMA,
          pltpu.SemaphoreType.DMA,
          pltpu.SemaphoreType.DMA,
          pltpu.SemaphoreType.DMA,
          pltpu.SemaphoreType.DMA,
          pltpu.SemaphoreType.DMA,
          pltpu.SemaphoreType.DMA,
          pltpu.SemaphoreType.DMA,
      ],
  )
  def _dists_call(out_hbm, bkt_hbm, sums2_hbm, cnt2_hbm, qs2_hbm, means_hbm,
                  buf0, buf1, buf2, buf3, buf4, buf5, idx_all, dist_a, dist_b,
                  work_v, cnt_v, inv_v, tbuf_v, qs_sh,
                  ld0, ld1, ld2, ld3, sc0, sc1, sc2, sc3, g0, g1):
    c = lax.axis_index("c")
    t = lax.axis_index("s")
    base = c * (_N // _NC) + t * _PT
    brow = t * _BT
    rbufs = (buf0, buf1, buf2)
    mbufs = (buf3, buf4, buf5)
    ldsems = (ld0, ld1, ld2)
    scsems = (sc0, sc1)
    gsems = (g0, g1, sc3)
    dists = (dist_a, dist_b)

    zeros16 = jnp.zeros((16,), jnp.float32)
    for g in range(8):
      dist_a[pl.ds(g * 16, 16)] = zeros16

    arow = pl.multiple_of(c * (_NB // _NS) + (t // 2) * (2 * _NCH), 8)
    half = (t % 2) * _NCH
    pltpu.sync_copy(bkt_hbm.at[pl.ds(arow, 2 * _NCH)], idx_all)

    pltpu.sync_copy(dist_a.at[pl.ds(0, _BT)], qs_sh.at[pl.ds(brow, _BT)])

    pltpu.sync_copy(sums2_hbm.at[0, pl.ds(brow, _BT)], work_v)
    pltpu.sync_copy(sums2_hbm.at[1, pl.ds(brow, _BT)], buf0.at[pl.ds(0, _BT)])
    pltpu.sync_copy(cnt2_hbm.at[0, pl.ds(brow, _BT)], cnt_v)
    pltpu.sync_copy(cnt2_hbm.at[1, pl.ds(brow, _BT)], dist_b.at[pl.ds(0, _BT)])
    for rg in range(_BT // 16):
      sl = pl.ds(rg * 16, 16)
      cv = cnt_v[sl] + dist_b[sl]
      inv_v[sl] = 1.0 / jnp.maximum(cv, 1.0)

    def _mrow(r, carry):
      ivec = plsc.load_gather(inv_v, [jnp.full((16,), 0, jnp.int32) + r])
      for g in range(8):
        sl = pl.ds(g * 16, 16)
        work_v[r, sl] = (work_v[r, sl] + buf0[r, sl]) * ivec
      return carry
    lax.fori_loop(0, _BT, _mrow, 0)
    pltpu.sync_copy(work_v, means_hbm.at[c, pl.ds(brow, _BT)])
    plsc.subcore_barrier()

    lid = lax.iota(jnp.int32, 16)

    def _compute(rows_v, mrows_v, dist_v):
      def _pgrp(pg, carry):
        for j in range(16):
          r = pg * 16 + j
          acc = jnp.zeros((16,), jnp.float32)
          for g in range(8):
            sl = pl.ds(g * 16, 16)
            d = rows_v[r, sl] - mrows_v[r, sl]
            acc = acc + d * d
          tbuf_v[j, pl.ds(0, 16)] = acc
        dvec = jnp.zeros((16,), jnp.float32)
        for dcol in range(16):
          col = jnp.full((16,), dcol, jnp.int32)
          dvec = dvec + plsc.load_gather(tbuf_v, [lid, col])
        d2 = dvec + 1e-12
        i = lax.bitcast_convert_type(d2, jnp.int32)
        i = 0x5F3759DF - lax.shift_right_logical(i, 1)
        y = lax.bitcast_convert_type(i, jnp.float32)
        y = y * (1.5 - 0.5 * d2 * y * y)
        y = y * (1.5 - 0.5 * d2 * y * y)
        y = y * (1.5 - 0.5 * d2 * y * y)
        dist_v[pl.ds(pg * 16, 16)] = d2 * y
        return carry
      lax.fori_loop(0, _C // 16, _pgrp, 0)

    g3h = [None] * _NCH
    l3h = [None] * _NCH
    s3h = [None] * _NCH
    for k in range(3):
      l3h[k] = pltpu.async_copy(out_hbm.at[pl.ds(base + k * _C, _C)],
                                rbufs[k], ldsems[k])
      g3h[k] = pltpu.async_copy(means_hbm.at[c].at[idx_all.at[half + k]],
                                mbufs[k], gsems[k])
    for k in range(_NCH):
      b = k % 3
      d = k & 1
      l3h[k].wait()
      g3h[k].wait()
      if k >= 2:
        s3h[k - 2].wait()
      _compute(rbufs[b], mbufs[b], dists[d])
      s3h[k] = pltpu.async_copy(dists[d], qs_sh.at[idx_all.at[half + k]],
                                scsems[d], add=True)
      if k + 3 < _NCH:
        l3h[k + 3] = pltpu.async_copy(
            out_hbm.at[pl.ds(base + (k + 3) * _C, _C)], rbufs[b], ldsems[b])
        g3h[k + 3] = pltpu.async_copy(means_hbm.at[c].at[idx_all.at[half + k + 3]],
                                      mbufs[b], gsems[b])
    s3h[_NCH - 2].wait()
    s3h[_NCH - 1].wait()
    plsc.subcore_barrier()

    pltpu.sync_copy(qs_sh.at[pl.ds(brow, _BT)], dist_a.at[pl.ds(0, _BT)])
    pltpu.sync_copy(dist_a.at[pl.ds(0, _BT)], qs2_hbm.at[c, pl.ds(brow, _BT)])

  return _dists_call


def _merge_body(qs2_ref, qs_ref):
    qs_ref[...] = qs2_ref[0:1, :] + qs2_ref[1:2, :]


_merge_call = pl.pallas_call(
    _merge_body,
    out_shape=jax.ShapeDtypeStruct((1, _NB), jnp.float32),
)


def kernel(inputs, W1, b1, W2, b2, a_lsh, b_lsh):
    a_pad = jnp.pad(a_lsh, ((0, 0), (0, _OUT - _NH)))
    bl_pad = jnp.pad(b_lsh, (0, _OUT - _NH)).reshape(1, _OUT)
    mix = jnp.asarray(_MIX)
    out, bkt2 = _tc_call(inputs, W1, b1.reshape(1, _HID), W2,
                         b2.reshape(1, _OUT), a_pad, bl_pad, mix)
    bkt = bkt2.reshape(_N // _C, _C)
    sums2, cnt2 = _build_sc_sums()(out, bkt)
    qs2, _ = _build_sc_dists()(out, bkt, sums2, cnt2)
    return _merge_call(qs2).reshape(_NB)


# speedup vs baseline: 1.0417x; 1.0417x over previous
import functools

import jax
import jax.numpy as jnp
import numpy as np
from jax import lax
from jax.experimental import pallas as pl
from jax.experimental.pallas import tpu as pltpu
from jax.experimental.pallas import tpu_sc as plsc

_N = 16384
_HID = 256
_OUT = 128
_NH = 16
_NB = 1024

_MIX = np.zeros((1, _OUT), dtype=np.int32)
_MIX[0, :_NH] = np.array(
    [73856093, 19349663, 83492791, 49979687, 67867967, 86028121,
     15485863, 32452843, 49979693, 67867979, 86028157, 15485867,
     2654435761 % (2**31 - 1), 40503, 2246822519 % (2**31 - 1),
     3266489917 % (2**31 - 1)], dtype=np.int32)


_BLK = 4096
_GRID = _N // _BLK


def _tc_body(x_ref, w1_ref, b1_ref, w2_ref, b2_ref, al_ref, bl_ref, pr_ref,
             out_ref, bkt_ref):
    x = x_ref[...]
    h = lax.dot_general(x, w1_ref[...], (((1,), (0,)), ((), ())),
                        preferred_element_type=jnp.float32)
    h = jnp.maximum(h + b1_ref[...], 0.0)
    out = lax.dot_general(h, w2_ref[...], (((1,), (0,)), ((), ())),
                          preferred_element_type=jnp.float32)
    out = out + b2_ref[...]
    out_ref[...] = out
    proj = lax.dot_general(out, al_ref[...], (((1,), (0,)), ((), ())),
                           preferred_element_type=jnp.float32)
    proj = proj + bl_ref[...]
    hcodes = jnp.floor(proj * 0.25).astype(jnp.int32)
    mixed = jnp.sum(hcodes * pr_ref[...], axis=1)
    bkt_ref[...] = lax.bitwise_and(mixed, 1023).reshape(_BLK // _C, _C)


_tc_call = pl.pallas_call(
    _tc_body,
    grid=(_GRID,),
    in_specs=[
        pl.BlockSpec((_BLK, _HID), lambda i: (i, 0)),
        pl.BlockSpec((_HID, _HID), lambda i: (0, 0)),
        pl.BlockSpec((1, _HID), lambda i: (0, 0)),
        pl.BlockSpec((_HID, _OUT), lambda i: (0, 0)),
        pl.BlockSpec((1, _OUT), lambda i: (0, 0)),
        pl.BlockSpec((_OUT, _OUT), lambda i: (0, 0)),
        pl.BlockSpec((1, _OUT), lambda i: (0, 0)),
        pl.BlockSpec((1, _OUT), lambda i: (0, 0)),
    ],
    out_specs=[
        pl.BlockSpec((_BLK, _OUT), lambda i: (i, 0)),
        pl.BlockSpec((_BLK // 128, 128), lambda i: (i, 0)),
    ],
    out_shape=[
        jax.ShapeDtypeStruct((_N, _OUT), jnp.float32),
        jax.ShapeDtypeStruct((_N // 128, 128), jnp.int32),
    ],
)


_NC = 2
_NS = 16
_PT = _N // (_NC * _NS)
_BT = _NB // _NS
_C = 128
_NCH = _PT // _C


@functools.cache
def _build_sc_sums():
  mesh = plsc.VectorSubcoreMesh(core_axis_name="c", subcore_axis_name="s",
                                num_cores=_NC, num_subcores=_NS)

  @functools.partial(
      pl.kernel,
      out_type=(jax.ShapeDtypeStruct((_NC, _NB, _OUT), jnp.float32),
                jax.ShapeDtypeStruct((_NC, _NB), jnp.float32)),
      mesh=mesh,
      compiler_params=pltpu.CompilerParams(needs_layout_passes=False),
      scratch_types=[
          pltpu.VMEM((_C, _OUT), jnp.float32),
          pltpu.VMEM((_C, _OUT), jnp.float32),
          pltpu.VMEM((_C, _OUT), jnp.float32),
          pltpu.VMEM((_C, _OUT), jnp.float32),
          pltpu.VMEM((2 * _NCH, _C), jnp.int32),
          pltpu.VMEM((_C,), jnp.float32),
          pltpu.VMEM((_BT, _OUT), jnp.float32),
          pltpu.VMEM((_BT,), jnp.float32),
          pltpu.VMEM_SHARED((_NB, _OUT), jnp.float32),
          pltpu.VMEM_SHARED((_NB,), jnp.float32),
          pltpu.SemaphoreType.DMA,
          pltpu.SemaphoreType.DMA,
          pltpu.SemaphoreType.DMA,
          pltpu.SemaphoreType.DMA,
          pltpu.SemaphoreType.DMA,
          pltpu.SemaphoreType.DMA,
          pltpu.SemaphoreType.DMA,
          pltpu.SemaphoreType.DMA,
          pltpu.SemaphoreType.DMA,
      ],
  )
  def _sums_call(out_hbm, bkt_hbm, sums2_hbm, cnt2_hbm,
                 buf0, buf1, buf2, buf3, idx_all, ones_v, work_v, cnt_v,
                 sums_sh, cnt_sh,
                 ld0, ld1, ld2, ld3, sc0, sc1, sc2, sc3, csem):
    c = lax.axis_index("c")
    t = lax.axis_index("s")
    base = c * (_N // _NC) + t * _PT
    brow = t * _BT
    bufs = (buf0, buf1, buf2, buf3)
    ldsems = (ld0, ld1, ld2, ld3)
    scsems = (sc0, sc1, sc2, sc3)

    zeros16 = jnp.zeros((16,), jnp.float32)
    for g in range(8):
      ones_v[pl.ds(g * 16, 16)] = jnp.ones((16,), jnp.float32)
    for g in range(_BT // 16):
      cnt_v[pl.ds(g * 16, 16)] = zeros16

    def _zrow(r, carry):
      for g in range(8):
        work_v[r, pl.ds(g * 16, 16)] = zeros16
      return carry
    lax.fori_loop(0, _BT, _zrow, 0)

    arow = pl.multiple_of(c * (_NB // _NS) + (t // 2) * (2 * _NCH), 8)
    half = (t % 2) * _NCH
    pltpu.sync_copy(bkt_hbm.at[pl.ds(arow, 2 * _NCH)], idx_all)

    pltpu.sync_copy(work_v, sums_sh.at[pl.ds(brow, _BT)])
    pltpu.sync_copy(cnt_v.at[pl.ds(0, _BT)], cnt_sh.at[pl.ds(brow, _BT)])
    plsc.subcore_barrier()

    ch = [pltpu.async_copy(ones_v, cnt_sh.at[idx_all.at[half + k]],
                           csem, add=True) for k in range(_NCH)]

    ldh = [pltpu.async_copy(out_hbm.at[pl.ds(base + k * _C, _C)],
                            bufs[k], ldsems[k]) for k in range(_NCH)]
    sch = [None] * _NCH
    for k in range(_NCH):
      ldh[k].wait()
      sch[k] = pltpu.async_copy(bufs[k], sums_sh.at[idx_all.at[half + k]],
                                scsems[k], add=True)
    for k in range(_NCH):
      sch[k].wait()
      ch[k].wait()
    plsc.subcore_barrier()

    pltpu.sync_copy(sums_sh.at[pl.ds(brow, _BT)], work_v)
    pltpu.sync_copy(cnt_sh.at[pl.ds(brow, _BT)], cnt_v)
    pltpu.sync_copy(work_v, sums2_hbm.at[c, pl.ds(brow, _BT)])
    pltpu.sync_copy(cnt_v, cnt2_hbm.at[c, pl.ds(brow, _BT)])

  return _sums_call


@functools.cache
def _build_sc_dists():
  mesh = plsc.VectorSubcoreMesh(core_axis_name="c", subcore_axis_name="s",
                                num_cores=_NC, num_subcores=_NS)

  @functools.partial(
      pl.kernel,
      out_type=jax.ShapeDtypeStruct((_NC, _NB), jnp.float32),
      mesh=mesh,
      compiler_params=pltpu.CompilerParams(needs_layout_passes=False),
      scratch_types=[
          pltpu.VMEM((_C, _OUT), jnp.float32),
          pltpu.VMEM((_C, _OUT), jnp.float32),
          pltpu.VMEM((_C, _OUT), jnp.float32),
          pltpu.VMEM((_C, _OUT), jnp.float32),
          pltpu.VMEM((_C, _OUT), jnp.float32),
          pltpu.VMEM((_C, _OUT), jnp.float32),
          pltpu.VMEM((2 * _NCH, _C), jnp.int32),
          pltpu.VMEM((_C,), jnp.float32),
          pltpu.VMEM((_C,), jnp.float32),
          pltpu.VMEM((_BT, _OUT), jnp.float32),
          pltpu.VMEM((_BT,), jnp.float32),
          pltpu.VMEM((_BT,), jnp.float32),
          pltpu.VMEM((16, 17), jnp.float32),
          pltpu.VMEM_SHARED((_NB, _OUT), jnp.float32),
          pltpu.VMEM_SHARED((_NB,), jnp.float32),
          pltpu.SemaphoreType.DMA,
          pltpu.SemaphoreType.DMA,
          pltpu.SemaphoreType.DMA,
          pltpu.SemaphoreType.DMA,
          pltpu.SemaphoreType.DMA,
          pltpu.SemaphoreType.DMA,
          pltpu.SemaphoreType.DMA,
          pltpu.SemaphoreType.DMA,
          pltpu.SemaphoreType.DMA,
          pltpu.SemaphoreType.DMA,
      ],
  )
  def _dists_call(out_hbm, bkt_hbm, sums2_hbm, cnt2_hbm, qs2_hbm,
                  buf0, buf1, buf2, buf3, buf4, buf5, idx_all, dist_a, dist_b,
                  work_v, cnt_v, inv_v, tbuf_v, means_sh, qs_sh,
                  ld0, ld1, ld2, ld3, sc0, sc1, sc2, sc3, g0, g1):
    c = lax.axis_index("c")
    t = lax.axis_index("s")
    base = c * (_N // _NC) + t * _PT
    brow = t * _BT
    rbufs = (buf0, buf1, buf2)
    mbufs = (buf3, buf4, buf5)
    ldsems = (ld0, ld1, ld2)
    scsems = (sc0, sc1)
    gsems = (g0, g1, sc3)
    dists = (dist_a, dist_b)

    zeros16 = jnp.zeros((16,), jnp.float32)
    for g in range(8):
      dist_a[pl.ds(g * 16, 16)] = zeros16

    arow = pl.multiple_of(c * (_NB // _NS) + (t // 2) * (2 * _NCH), 8)
    half = (t % 2) * _NCH
    pltpu.sync_copy(bkt_hbm.at[pl.ds(arow, 2 * _NCH)], idx_all)

    pltpu.sync_copy(dist_a.at[pl.ds(0, _BT)], qs_sh.at[pl.ds(brow, _BT)])

    l3h = [None] * _NCH
    for k in range(3):
      l3h[k] = pltpu.async_copy(out_hbm.at[pl.ds(base + k * _C, _C)],
                                rbufs[k], ldsems[k])

    ph = (pltpu.async_copy(sums2_hbm.at[0, pl.ds(brow, _BT)], work_v, ld3),
          pltpu.async_copy(sums2_hbm.at[1, pl.ds(brow, _BT)],
                           buf3.at[pl.ds(0, _BT)], sc2),
          pltpu.async_copy(cnt2_hbm.at[0, pl.ds(brow, _BT)], cnt_v, g0),
          pltpu.async_copy(cnt2_hbm.at[1, pl.ds(brow, _BT)],
                           dist_b.at[pl.ds(0, _BT)], g1))
    for h in ph:
      h.wait()
    for rg in range(_BT // 16):
      sl = pl.ds(rg * 16, 16)
      cv = cnt_v[sl] + dist_b[sl]
      inv_v[sl] = 1.0 / jnp.maximum(cv, 1.0)

    def _mrow(r, carry):
      ivec = plsc.load_gather(inv_v, [jnp.full((16,), 0, jnp.int32) + r])
      for g in range(8):
        sl = pl.ds(g * 16, 16)
        work_v[r, sl] = (work_v[r, sl] + buf3[r, sl]) * ivec
      return carry
    lax.fori_loop(0, _BT, _mrow, 0)
    pltpu.sync_copy(work_v, means_sh.at[pl.ds(brow, _BT)])
    plsc.subcore_barrier()

    lid = lax.iota(jnp.int32, 16)

    def _compute(rows_v, mrows_v, dist_v):
      def _pgrp(pg, carry):
        for j in range(16):
          r = pg * 16 + j
          acc = jnp.zeros((16,), jnp.float32)
          for g in range(8):
            sl = pl.ds(g * 16, 16)
            d = rows_v[r, sl] - mrows_v[r, sl]
            acc = acc + d * d
          tbuf_v[j, pl.ds(0, 16)] = acc
        dvec = jnp.zeros((16,), jnp.float32)
        for dcol in range(16):
          col = jnp.full((16,), dcol, jnp.int32)
          dvec = dvec + plsc.load_gather(tbuf_v, [lid, col])
        d2 = dvec + 1e-12
        i = lax.bitcast_convert_type(d2, jnp.int32)
        i = 0x5F3759DF - lax.shift_right_logical(i, 1)
        y = lax.bitcast_convert_type(i, jnp.float32)
        y = y * (1.5 - 0.5 * d2 * y * y)
        y = y * (1.5 - 0.5 * d2 * y * y)
        y = y * (1.5 - 0.5 * d2 * y * y)
        dist_v[pl.ds(pg * 16, 16)] = d2 * y
        return carry
      lax.fori_loop(0, _C // 16, _pgrp, 0)

    g3h = [None] * _NCH
    s3h = [None] * _NCH
    for k in range(3):
      g3h[k] = pltpu.async_copy(means_sh.at[idx_all.at[half + k]],
                                mbufs[k], gsems[k])
    for k in range(_NCH):
      b = k % 3
      d = k & 1
      l3h[k].wait()
      g3h[k].wait()
      if k >= 2:
        s3h[k - 2].wait()
      _compute(rbufs[b], mbufs[b], dists[d])
      s3h[k] = pltpu.async_copy(dists[d], qs_sh.at[idx_all.at[half + k]],
                                scsems[d], add=True)
      if k + 3 < _NCH:
        l3h[k + 3] = pltpu.async_copy(
            out_hbm.at[pl.ds(base + (k + 3) * _C, _C)], rbufs[b], ldsems[b])
        g3h[k + 3] = pltpu.async_copy(means_sh.at[idx_all.at[half + k + 3]],
                                      mbufs[b], gsems[b])
    s3h[_NCH - 2].wait()
    s3h[_NCH - 1].wait()
    plsc.subcore_barrier()

    pltpu.sync_copy(qs_sh.at[pl.ds(brow, _BT)], dist_a.at[pl.ds(0, _BT)])
    pltpu.sync_copy(dist_a.at[pl.ds(0, _BT)], qs2_hbm.at[c, pl.ds(brow, _BT)])

  return _dists_call


def _merge_body(qs2_ref, qs_ref):
    qs_ref[...] = qs2_ref[0:1, :] + qs2_ref[1:2, :]


_merge_call = pl.pallas_call(
    _merge_body,
    out_shape=jax.ShapeDtypeStruct((1, _NB), jnp.float32),
)


def kernel(inputs, W1, b1, W2, b2, a_lsh, b_lsh):
    a_pad = jnp.pad(a_lsh, ((0, 0), (0, _OUT - _NH)))
    bl_pad = jnp.pad(b_lsh, (0, _OUT - _NH)).reshape(1, _OUT)
    mix = jnp.asarray(_MIX)
    out, bkt2 = _tc_call(inputs, W1, b1.reshape(1, _HID), W2,
                         b2.reshape(1, _OUT), a_pad, bl_pad, mix)
    bkt = bkt2.reshape(_N // _C, _C)
    sums2, cnt2 = _build_sc_sums()(out, bkt)
    qs2 = _build_sc_dists()(out, bkt, sums2, cnt2)
    return _merge_call(qs2).reshape(_NB)
